# Initial kernel scaffold; baseline (speedup 1.0000x reference)
import jax
import jax.numpy as jnp
from jax.experimental import pallas as pl


def kernel(x, table):
    raise NotImplementedError("write your pallas kernel here")


# speedup vs baseline: 1.4800x; 1.4800x over previous
import functools

import jax
import jax.numpy as jnp
from jax import lax
from jax.experimental import pallas as pl
from jax.experimental.pallas import tpu as pltpu
from jax.experimental.pallas import tpu_sc as plsc

EMB = 32
B_TOTAL = 4096 * 200
NUM_WORKERS = 32
BPW = B_TOTAL // NUM_WORKERS
CHUNK = 1600
NCHUNK = BPW // CHUNK


def _emb_body(x_hbm, table_hbm, out_hbm, idx_v, rows_v, sem):
    wid = lax.axis_index("s") * 2 + lax.axis_index("c")
    base = wid * BPW

    def body(j, carry):
        off = base + j * CHUNK
        pltpu.sync_copy(x_hbm.at[pl.ds(off, CHUNK)], idx_v)
        pltpu.async_copy(table_hbm.at[idx_v], rows_v, sem).wait()
        pltpu.sync_copy(rows_v, out_hbm.at[pl.ds(off, CHUNK)])
        return carry

    lax.fori_loop(0, NCHUNK, body, 0)


def kernel(x, table):
    xf = x.reshape(-1)
    mesh = plsc.VectorSubcoreMesh(core_axis_name="c", subcore_axis_name="s")
    run = functools.partial(
        pl.kernel,
        mesh=mesh,
        out_type=jax.ShapeDtypeStruct((B_TOTAL, EMB), jnp.float32),
        scratch_types=[
            pltpu.VMEM((CHUNK,), jnp.int32),
            pltpu.VMEM((CHUNK, EMB), jnp.float32),
            pltpu.SemaphoreType.DMA,
        ],
        compiler_params=pltpu.CompilerParams(use_tc_tiling_on_sc=False),
    )(_emb_body)
    out = run(xf, table)
    return out.reshape(x.shape[0], x.shape[1], EMB)


# speedup vs baseline: 1.5004x; 1.0138x over previous
import functools

import jax
import jax.numpy as jnp
from jax import lax
from jax.experimental import pallas as pl
from jax.experimental.pallas import tpu as pltpu
from jax.experimental.pallas import tpu_sc as plsc

EMB = 32
B_TOTAL = 4096 * 200
NUM_WORKERS = 32
BPW = B_TOTAL // NUM_WORKERS
CHUNK = 1280
NCHUNK = BPW // CHUNK


def _emb_body(x_hbm, table_hbm, out_hbm, idx_v, rows0, rows1,
              semg0, semg1, semw0, semw1):
    wid = lax.axis_index("s") * 2 + lax.axis_index("c")
    base = wid * BPW
    pltpu.sync_copy(x_hbm.at[pl.ds(base, BPW)], idx_v)

    rows = (rows0, rows1)
    semg = (semg0, semg1)
    semw = (semw0, semw1)

    def issue_gather(j, buf):
        return pltpu.async_copy(
            table_hbm.at[idx_v.at[pl.ds(j * CHUNK, CHUNK)]],
            rows[buf], semg[buf])

    def issue_wb(j, buf):
        return pltpu.async_copy(
            rows[buf], out_hbm.at[pl.ds(base + j * CHUNK, CHUNK)], semw[buf])

    wb = [None, None]
    gh = [None, None]
    gh[0] = issue_gather(0, 0)
    for j in range(NCHUNK):
        buf = j % 2
        nbuf = 1 - buf
        if j + 1 < NCHUNK:
            if wb[nbuf] is not None:
                wb[nbuf].wait()
            gh[nbuf] = issue_gather(j + 1, nbuf)
        gh[buf].wait()
        wb[buf] = issue_wb(j, buf)
    wb[0].wait()
    wb[1].wait()


def kernel(x, table):
    xf = x.reshape(-1)
    mesh = plsc.VectorSubcoreMesh(core_axis_name="c", subcore_axis_name="s")
    run = functools.partial(
        pl.kernel,
        mesh=mesh,
        out_type=jax.ShapeDtypeStruct((B_TOTAL, EMB), jnp.float32),
        scratch_types=[
            pltpu.VMEM((BPW,), jnp.int32),
            pltpu.VMEM((CHUNK, EMB), jnp.float32),
            pltpu.VMEM((CHUNK, EMB), jnp.float32),
            pltpu.SemaphoreType.DMA,
            pltpu.SemaphoreType.DMA,
            pltpu.SemaphoreType.DMA,
            pltpu.SemaphoreType.DMA,
        ],
        compiler_params=pltpu.CompilerParams(use_tc_tiling_on_sc=False),
    )(_emb_body)
    out = run(xf, table)
    return out.reshape(x.shape[0], x.shape[1], EMB)
